# Initial kernel scaffold; baseline (speedup 1.0000x reference)
import jax
import jax.numpy as jnp
from jax.experimental import pallas as pl


def kernel(x, edge_index, W1, W2):
    raise NotImplementedError("write your pallas kernel here")


# speedup vs baseline: 21.8006x; 21.8006x over previous
import functools

import jax
import jax.numpy as jnp
from jax import lax
from jax.experimental import pallas as pl
from jax.experimental.pallas import tpu as pltpu
from jax.experimental.pallas import tpu_sc as plsc

N = 10000
E = 320000
NC, NS = 2, 16
NW = NC * NS
EW = E // NW
CHUNK = 80
NCH = EW // CHUNK
RPT = N // NS

_MESH = plsc.VectorSubcoreMesh(
    core_axis_name="c", subcore_axis_name="s", num_cores=NC, num_subcores=NS
)


def _hist_body(row_hbm, zeros_hbm, deg0_out, deg1_out, idx_v, ones_v, acc_sh,
               sem):
    c = lax.axis_index("c")
    s = lax.axis_index("s")
    wid = c * NS + s

    @pl.when(s == 0)
    def _():
        pltpu.sync_copy(zeros_hbm, acc_sh)

    for i in range(CHUNK // 16):
        ones_v[pl.ds(i * 16, 16)] = jnp.ones((16,), jnp.float32)
    pltpu.sync_copy(row_hbm.at[wid], idx_v)
    plsc.subcore_barrier()

    def step(j, carry):
        pltpu.sync_copy(ones_v, acc_sh.at[idx_v.at[j]], add=True)
        return carry

    lax.fori_loop(0, NCH, step, 0)
    plsc.subcore_barrier()

    @pl.when((s == 0) & (c == 0))
    def _():
        pltpu.sync_copy(acc_sh, deg0_out)

    @pl.when((s == 0) & (c == 1))
    def _():
        pltpu.sync_copy(acc_sh, deg1_out)


def _histogram(row_r, zeros_n):
    kern = pl.kernel(
        _hist_body,
        out_type=(jax.ShapeDtypeStruct((N,), jnp.float32),
                  jax.ShapeDtypeStruct((N,), jnp.float32)),
        mesh=_MESH,
        scratch_types=[
            pltpu.VMEM((NCH, CHUNK), jnp.int32),
            pltpu.VMEM((CHUNK,), jnp.float32),
            pltpu.VMEM_SHARED((N,), jnp.float32),
            pltpu.SemaphoreType.DMA,
        ],
    )
    return kern(row_r, zeros_n)


_STRIPE = 624


def _scat_body(g_hbm, col_hbm, row_hbm, zeros_hbm, acc_out, colv, rowv,
               rows_v, acc_sh, sem, *, d):
    c = lax.axis_index("c")
    s = lax.axis_index("s")
    wid = c * NS + s

    off = s * _STRIPE
    pltpu.sync_copy(zeros_hbm.at[pl.ds(off, _STRIPE)],
                    acc_sh.at[pl.ds(off, _STRIPE)])

    @pl.when(s == NS - 1)
    def _():
        tail = (NS - 1) * _STRIPE + _STRIPE
        pltpu.sync_copy(zeros_hbm.at[pl.ds(tail, N - tail)],
                        acc_sh.at[pl.ds(tail, N - tail)])

    pltpu.sync_copy(col_hbm.at[wid], colv)
    pltpu.sync_copy(row_hbm.at[wid], rowv)
    plsc.subcore_barrier()

    def step(j, carry):
        pltpu.async_copy(g_hbm.at[colv.at[j]], rows_v, sem).wait()
        pltpu.sync_copy(rows_v, acc_sh.at[rowv.at[j]], add=True)
        return carry

    lax.fori_loop(0, NCH, step, 0)
    plsc.subcore_barrier()
    pltpu.sync_copy(acc_sh.at[pl.ds(off, _STRIPE)],
                    acc_out.at[c, pl.ds(off, _STRIPE)])

    @pl.when(s == NS - 1)
    def _():
        tail = (NS - 1) * _STRIPE + _STRIPE
        pltpu.sync_copy(acc_sh.at[pl.ds(tail, N - tail)],
                        acc_out.at[c, pl.ds(tail, N - tail)])


def _scatter(g, col_r, row_r, zeros_nd, d):
    kern = pl.kernel(
        functools.partial(_scat_body, d=d),
        out_type=jax.ShapeDtypeStruct((NC, N, d), jnp.float32),
        mesh=_MESH,
        scratch_types=[
            pltpu.VMEM((NCH, CHUNK), jnp.int32),
            pltpu.VMEM((NCH, CHUNK), jnp.int32),
            pltpu.VMEM((CHUNK, d), jnp.float32),
            pltpu.VMEM_SHARED((N, d), jnp.float32),
            pltpu.SemaphoreType.DMA,
        ],
        compiler_params=pltpu.CompilerParams(use_tc_tiling_on_sc=False),
    )
    return kern(g, col_r, row_r, zeros_nd)


_RT = 400


def _mm1_body(x_ref, w_ref, o_ref):
    o_ref[...] = jnp.dot(x_ref[...], w_ref[...],
                         preferred_element_type=jnp.float32)


def _scale_body(d0_ref, d1_ref, h_ref, dis_ref, g_ref):
    deg = d0_ref[...] + d1_ref[...] + 1.0
    dis = lax.rsqrt(deg)
    dis_ref[...] = dis
    g_ref[...] = dis * h_ref[...]


def _l2_body(a0_ref, a1_ref, g_ref, dis_ref, w_ref, o_ref):
    s = dis_ref[...] * (a0_ref[...] + a1_ref[...] + g_ref[...])
    h = jnp.maximum(s, 0.0)
    o_ref[...] = dis_ref[...] * jnp.dot(h, w_ref[...],
                                        preferred_element_type=jnp.float32)


def _fin_body(a0_ref, a1_ref, g_ref, dis_ref, o_ref):
    s = dis_ref[...] * (a0_ref[...] + a1_ref[...] + g_ref[...])
    nrm = jnp.sqrt(jnp.sum(s * s, axis=1, keepdims=True))
    o_ref[...] = s / jnp.maximum(nrm, 1e-7)


def _rows(d):
    return pl.BlockSpec((_RT, d), lambda i: (i, 0))


def kernel(x, edge_index, W1, W2):
    d1 = W1.shape[1]
    d2 = W2.shape[1]
    row_r = edge_index[0].reshape(NW, NCH, CHUNK)
    col_r = edge_index[1].reshape(NW, NCH, CHUNK)
    zeros_n = jnp.zeros((N,), jnp.float32)
    zeros_n1 = jnp.zeros((N, d1), jnp.float32)
    zeros_n2 = jnp.zeros((N, d2), jnp.float32)

    deg0, deg1 = _histogram(row_r, zeros_n)

    h1 = pl.pallas_call(
        _mm1_body,
        grid=(N // _RT,),
        in_specs=[pl.BlockSpec((_RT, x.shape[1]), lambda i: (i, 0)),
                  pl.BlockSpec((x.shape[1], d1), lambda i: (0, 0))],
        out_specs=_rows(d1),
        out_shape=jax.ShapeDtypeStruct((N, d1), jnp.float32),
    )(x, W1)

    dis, g1 = pl.pallas_call(
        _scale_body,
        grid=(N // _RT,),
        in_specs=[_rows(1), _rows(1), _rows(d1)],
        out_specs=[_rows(1), _rows(d1)],
        out_shape=[jax.ShapeDtypeStruct((N, 1), jnp.float32),
                   jax.ShapeDtypeStruct((N, d1), jnp.float32)],
    )(deg0[:, None], deg1[:, None], h1)

    acc1 = _scatter(g1, col_r, row_r, zeros_n1, d1)

    g2 = pl.pallas_call(
        _l2_body,
        grid=(N // _RT,),
        in_specs=[_rows(d1), _rows(d1), _rows(d1), _rows(1),
                  pl.BlockSpec((d1, d2), lambda i: (0, 0))],
        out_specs=_rows(d2),
        out_shape=jax.ShapeDtypeStruct((N, d2), jnp.float32),
    )(acc1[0], acc1[1], g1, dis, W2)

    acc2 = _scatter(g2, col_r, row_r, zeros_n2, d2)

    out = pl.pallas_call(
        _fin_body,
        grid=(N // _RT,),
        in_specs=[_rows(d2), _rows(d2), _rows(d2), _rows(1)],
        out_specs=_rows(d2),
        out_shape=jax.ShapeDtypeStruct((N, d2), jnp.float32),
    )(acc2[0], acc2[1], g2, dis)
    return out


# speedup vs baseline: 36.0141x; 1.6520x over previous
import functools

import jax
import jax.numpy as jnp
from jax import lax
from jax.experimental import pallas as pl
from jax.experimental.pallas import tpu as pltpu
from jax.experimental.pallas import tpu_sc as plsc

N = 10000
E = 320000
NC, NS = 2, 16
NW = NC * NS
EW = E // NW
CHUNK = 80
NCH = EW // CHUNK
CH_S = 125
NCH_S = EW // CH_S
NBUF = 4
RPT = N // NS

_MESH = plsc.VectorSubcoreMesh(
    core_axis_name="c", subcore_axis_name="s", num_cores=NC, num_subcores=NS
)


def _hist_body(row_hbm, zeros_hbm, deg0_out, deg1_out, idx_v, ones_v, acc_sh,
               sem):
    c = lax.axis_index("c")
    s = lax.axis_index("s")
    wid = c * NS + s

    @pl.when(s == 0)
    def _():
        pltpu.sync_copy(zeros_hbm, acc_sh)

    for i in range(CHUNK // 16):
        ones_v[pl.ds(i * 16, 16)] = jnp.ones((16,), jnp.float32)
    pltpu.sync_copy(row_hbm.at[wid], idx_v)
    plsc.subcore_barrier()

    def step(j, carry):
        pltpu.sync_copy(ones_v, acc_sh.at[idx_v.at[j]], add=True)
        return carry

    lax.fori_loop(0, NCH, step, 0)
    plsc.subcore_barrier()

    @pl.when((s == 0) & (c == 0))
    def _():
        pltpu.sync_copy(acc_sh, deg0_out)

    @pl.when((s == 0) & (c == 1))
    def _():
        pltpu.sync_copy(acc_sh, deg1_out)


def _histogram(row_r, zeros_n):
    kern = pl.kernel(
        _hist_body,
        out_type=(jax.ShapeDtypeStruct((N,), jnp.float32),
                  jax.ShapeDtypeStruct((N,), jnp.float32)),
        mesh=_MESH,
        scratch_types=[
            pltpu.VMEM((NCH, CHUNK), jnp.int32),
            pltpu.VMEM((CHUNK,), jnp.float32),
            pltpu.VMEM_SHARED((N,), jnp.float32),
            pltpu.SemaphoreType.DMA,
        ],
    )
    return kern(row_r, zeros_n)


_STRIPE = 624


def _scat_body(g_hbm, col_hbm, row_hbm, zeros_hbm, acc_out, colv, rowv,
               b0, b1, b2, b3, acc_sh, s0, s1, s2, s3, *, d):
    c = lax.axis_index("c")
    s = lax.axis_index("s")
    wid = c * NS + s
    bufs = (b0, b1, b2, b3)
    sems = (s0, s1, s2, s3)

    off = s * _STRIPE
    pltpu.sync_copy(zeros_hbm.at[pl.ds(off, _STRIPE)],
                    acc_sh.at[pl.ds(off, _STRIPE)])

    @pl.when(s == NS - 1)
    def _():
        tail = (NS - 1) * _STRIPE + _STRIPE
        pltpu.sync_copy(zeros_hbm.at[pl.ds(tail, N - tail)],
                        acc_sh.at[pl.ds(tail, N - tail)])

    pltpu.sync_copy(col_hbm.at[wid], colv)
    pltpu.sync_copy(row_hbm.at[wid], rowv)
    plsc.subcore_barrier()

    for b in range(NBUF):
        pltpu.async_copy(g_hbm.at[colv.at[b]], bufs[b], sems[b])

    def outer(t, carry):
        for b in range(NBUF):
            j = t * NBUF + b
            pltpu.make_async_copy(g_hbm.at[pl.ds(0, CH_S)], bufs[b],
                                  sems[b]).wait()
            pltpu.sync_copy(bufs[b], acc_sh.at[rowv.at[j]], add=True)

            @pl.when(j + NBUF < NCH_S)
            def _():
                pltpu.async_copy(g_hbm.at[colv.at[j + NBUF]], bufs[b],
                                 sems[b])
        return carry

    lax.fori_loop(0, NCH_S // NBUF, outer, 0)
    plsc.subcore_barrier()
    pltpu.sync_copy(acc_sh.at[pl.ds(off, _STRIPE)],
                    acc_out.at[c, pl.ds(off, _STRIPE)])

    @pl.when(s == NS - 1)
    def _():
        tail = (NS - 1) * _STRIPE + _STRIPE
        pltpu.sync_copy(acc_sh.at[pl.ds(tail, N - tail)],
                        acc_out.at[c, pl.ds(tail, N - tail)])


def _scatter(g, col_r, row_r, zeros_nd, d):
    kern = pl.kernel(
        functools.partial(_scat_body, d=d),
        out_type=jax.ShapeDtypeStruct((NC, N, d), jnp.float32),
        mesh=_MESH,
        scratch_types=[
            pltpu.VMEM((NCH_S, CH_S), jnp.int32),
            pltpu.VMEM((NCH_S, CH_S), jnp.int32),
            pltpu.VMEM((CH_S, d), jnp.float32),
            pltpu.VMEM((CH_S, d), jnp.float32),
            pltpu.VMEM((CH_S, d), jnp.float32),
            pltpu.VMEM((CH_S, d), jnp.float32),
            pltpu.VMEM_SHARED((N, d), jnp.float32),
            pltpu.SemaphoreType.DMA,
            pltpu.SemaphoreType.DMA,
            pltpu.SemaphoreType.DMA,
            pltpu.SemaphoreType.DMA,
        ],
        compiler_params=pltpu.CompilerParams(use_tc_tiling_on_sc=False),
    )
    return kern(g, col_r, row_r, zeros_nd)


_RT = 400


def _mm1_body(x_ref, w_ref, o_ref):
    o_ref[...] = jnp.dot(x_ref[...], w_ref[...],
                         preferred_element_type=jnp.float32)


def _scale_body(d0_ref, d1_ref, h_ref, dis_ref, g_ref):
    deg = d0_ref[...] + d1_ref[...] + 1.0
    dis = lax.rsqrt(deg)
    dis_ref[...] = dis
    g_ref[...] = dis * h_ref[...]


def _l2_body(a0_ref, a1_ref, g_ref, dis_ref, w_ref, o_ref):
    s = dis_ref[...] * (a0_ref[...] + a1_ref[...] + g_ref[...])
    h = jnp.maximum(s, 0.0)
    o_ref[...] = dis_ref[...] * jnp.dot(h, w_ref[...],
                                        preferred_element_type=jnp.float32)


def _fin_body(a0_ref, a1_ref, g_ref, dis_ref, o_ref):
    s = dis_ref[...] * (a0_ref[...] + a1_ref[...] + g_ref[...])
    nrm = jnp.sqrt(jnp.sum(s * s, axis=1, keepdims=True))
    o_ref[...] = s / jnp.maximum(nrm, 1e-7)


def _rows(d):
    return pl.BlockSpec((_RT, d), lambda i: (i, 0))


def kernel(x, edge_index, W1, W2):
    d1 = W1.shape[1]
    d2 = W2.shape[1]
    row_r = edge_index[0].reshape(NW, NCH, CHUNK)
    col_r = edge_index[1].reshape(NW, NCH, CHUNK)
    row_s = edge_index[0].reshape(NW, NCH_S, CH_S)
    col_s = edge_index[1].reshape(NW, NCH_S, CH_S)
    zeros_n = jnp.zeros((N,), jnp.float32)
    zeros_n1 = jnp.zeros((N, d1), jnp.float32)
    zeros_n2 = jnp.zeros((N, d2), jnp.float32)

    deg0, deg1 = _histogram(row_r, zeros_n)

    h1 = pl.pallas_call(
        _mm1_body,
        grid=(N // _RT,),
        in_specs=[pl.BlockSpec((_RT, x.shape[1]), lambda i: (i, 0)),
                  pl.BlockSpec((x.shape[1], d1), lambda i: (0, 0))],
        out_specs=_rows(d1),
        out_shape=jax.ShapeDtypeStruct((N, d1), jnp.float32),
    )(x, W1)

    dis, g1 = pl.pallas_call(
        _scale_body,
        grid=(N // _RT,),
        in_specs=[_rows(1), _rows(1), _rows(d1)],
        out_specs=[_rows(1), _rows(d1)],
        out_shape=[jax.ShapeDtypeStruct((N, 1), jnp.float32),
                   jax.ShapeDtypeStruct((N, d1), jnp.float32)],
    )(deg0[:, None], deg1[:, None], h1)

    acc1 = _scatter(g1, col_s, row_s, zeros_n1, d1)

    g2 = pl.pallas_call(
        _l2_body,
        grid=(N // _RT,),
        in_specs=[_rows(d1), _rows(d1), _rows(d1), _rows(1),
                  pl.BlockSpec((d1, d2), lambda i: (0, 0))],
        out_specs=_rows(d2),
        out_shape=jax.ShapeDtypeStruct((N, d2), jnp.float32),
    )(acc1[0], acc1[1], g1, dis, W2)

    acc2 = _scatter(g2, col_s, row_s, zeros_n2, d2)

    out = pl.pallas_call(
        _fin_body,
        grid=(N // _RT,),
        in_specs=[_rows(d2), _rows(d2), _rows(d2), _rows(1)],
        out_specs=_rows(d2),
        out_shape=jax.ShapeDtypeStruct((N, d2), jnp.float32),
    )(acc2[0], acc2[1], g2, dis)
    return out
